# Initial kernel scaffold; baseline (speedup 1.0000x reference)
import jax
import jax.numpy as jnp
from jax.experimental import pallas as pl


def kernel(vertex_features, faces):
    raise NotImplementedError("write your pallas kernel here")


# speedup vs baseline: 3.4610x; 3.4610x over previous
import functools

import jax
import jax.numpy as jnp
from jax import lax
from jax.experimental import pallas as pl
from jax.experimental.pallas import tpu as pltpu
from jax.experimental.pallas import tpu_sc as plsc

NUM_CORES = 2
NUM_SUBCORES = 16
NUM_TILES = NUM_CORES * NUM_SUBCORES

NV, D = 100000, 128
NF = 200000
NF_PAD = 204800
FACES_PER_TILE = NF_PAD // NUM_TILES
CHUNK_FACES = 128
N_CHUNKS = FACES_PER_TILE // CHUNK_FACES
ROWS_PER_CHUNK = 3 * CHUNK_FACES
GATHERS_PER_CHUNK = ROWS_PER_CHUNK // 128
IDX_PER_TILE = 3 * FACES_PER_TILE
LANES = 16
GROUPS = D // LANES

_sc_mesh = plsc.VectorSubcoreMesh(core_axis_name="c", subcore_axis_name="s")


@functools.partial(
    pl.kernel,
    mesh=_sc_mesh,
    out_type=jax.ShapeDtypeStruct((NF_PAD, 3 * LANES), jnp.float32),
    scratch_types=[
        pltpu.VMEM((IDX_PER_TILE,), jnp.int32),
        pltpu.VMEM((ROWS_PER_CHUNK, D), jnp.float32),
        pltpu.VMEM((CHUNK_FACES, 3 * LANES), jnp.float32),
        pltpu.SemaphoreType.DMA,
    ],
)
def _sc_pair_partials(table_hbm, idx_hbm, out_hbm, idx_v, rows_v, out_v, sem):
    wid = lax.axis_index("s") * NUM_CORES + lax.axis_index("c")
    pltpu.sync_copy(idx_hbm.at[pl.ds(wid * IDX_PER_TILE, IDX_PER_TILE)], idx_v)

    def chunk_body(c, carry):
        cps = [
            pltpu.async_copy(
                table_hbm.at[idx_v.at[pl.ds(c * ROWS_PER_CHUNK + j * 128, 128)]],
                rows_v.at[pl.ds(j * 128, 128)],
                sem,
            )
            for j in range(GATHERS_PER_CHUNK)
        ]
        for cp in cps:
            cp.wait()

        def face_body(f, fc):
            r = 3 * f
            zero = jnp.zeros((LANES,), jnp.float32)
            acc01, acc02, acc21 = zero, zero, zero
            for g in range(GROUPS):
                sl = pl.ds(g * LANES, LANES)
                va = rows_v[r, sl]
                vb = rows_v[r + 1, sl]
                vc = rows_v[r + 2, sl]
                d01 = va - vb
                d02 = va - vc
                d21 = vc - vb
                acc01 = acc01 + d01 * d01
                acc02 = acc02 + d02 * d02
                acc21 = acc21 + d21 * d21
            out_v[f, pl.ds(0, LANES)] = acc01
            out_v[f, pl.ds(LANES, LANES)] = acc02
            out_v[f, pl.ds(2 * LANES, LANES)] = acc21
            return fc

        lax.fori_loop(0, CHUNK_FACES, face_body, 0)
        out_base = wid * FACES_PER_TILE + c * CHUNK_FACES
        pltpu.sync_copy(out_v, out_hbm.at[pl.ds(out_base, CHUNK_FACES)])
        return carry

    lax.fori_loop(0, N_CHUNKS, chunk_body, 0)


_TC_BLK = 8192
_TC_GRID = NF_PAD // _TC_BLK


def _tc_reduce_body(x_ref, o_ref):
    i = pl.program_id(0)
    x = x_ref[...]
    d2a = jnp.sum(x[:, 0:LANES], axis=1)
    d2b = jnp.sum(x[:, LANES:2 * LANES], axis=1)
    d2c = jnp.sum(x[:, 2 * LANES:3 * LANES], axis=1)
    s = jnp.sum(jnp.sqrt(d2a)) + jnp.sum(jnp.sqrt(d2b)) + jnp.sum(jnp.sqrt(d2c))

    @pl.when(i == 0)
    def _init():
        o_ref[0, 0] = s

    @pl.when(i != 0)
    def _acc():
        o_ref[0, 0] = o_ref[0, 0] + s


_tc_reduce = pl.pallas_call(
    _tc_reduce_body,
    grid=(_TC_GRID,),
    in_specs=[pl.BlockSpec((_TC_BLK, 3 * LANES), lambda i: (i, 0))],
    out_specs=pl.BlockSpec(memory_space=pltpu.SMEM),
    out_shape=jax.ShapeDtypeStruct((1, 1), jnp.float32),
)


def kernel(vertex_features, faces):
    table = vertex_features.reshape(NV, D)
    flat_idx = faces.reshape(-1)
    pad = jnp.zeros(((NF_PAD - NF) * 3,), jnp.int32)
    idx_flat = jnp.concatenate([flat_idx, pad])
    partials = _sc_pair_partials(table, idx_flat)
    total = _tc_reduce(partials)
    return total.reshape(1)


# speedup vs baseline: 4.0629x; 1.1739x over previous
import functools

import jax
import jax.numpy as jnp
from jax import lax
from jax.experimental import pallas as pl
from jax.experimental.pallas import tpu as pltpu
from jax.experimental.pallas import tpu_sc as plsc

NUM_CORES = 2
NUM_SUBCORES = 16
NUM_TILES = NUM_CORES * NUM_SUBCORES

NV, D = 100000, 128
NF = 200000
NF_PAD = 204800
FACES_PER_TILE = NF_PAD // NUM_TILES
CHUNK_FACES = 64
N_CHUNKS = FACES_PER_TILE // CHUNK_FACES
ROWS_PER_CHUNK = 3 * CHUNK_FACES
GATHERS_PER_CHUNK = 2
GATHER_ROWS = ROWS_PER_CHUNK // GATHERS_PER_CHUNK
IDX_PER_TILE = 3 * FACES_PER_TILE
IDX_STAGE = IDX_PER_TILE + 2 * ROWS_PER_CHUNK
IDX_HBM_LEN = (NUM_TILES - 1) * IDX_PER_TILE + IDX_STAGE
LANES = 16
GROUPS = D // LANES

_sc_mesh = plsc.VectorSubcoreMesh(core_axis_name="c", subcore_axis_name="s")


@functools.partial(
    pl.kernel,
    mesh=_sc_mesh,
    out_type=jax.ShapeDtypeStruct((NF_PAD, 3 * LANES), jnp.float32),
    scratch_types=[
        pltpu.VMEM((IDX_STAGE,), jnp.int32),
        pltpu.VMEM((ROWS_PER_CHUNK, D), jnp.float32),
        pltpu.VMEM((ROWS_PER_CHUNK, D), jnp.float32),
        pltpu.VMEM((CHUNK_FACES, 3 * LANES), jnp.float32),
        pltpu.VMEM((CHUNK_FACES, 3 * LANES), jnp.float32),
        pltpu.SemaphoreType.DMA,
        pltpu.SemaphoreType.DMA,
        pltpu.SemaphoreType.DMA,
        pltpu.SemaphoreType.DMA,
    ],
)
def _sc_pair_partials(table_hbm, idx_hbm, out_hbm, idx_v, rows_v0, rows_v1,
                      out_v0, out_v1, sg0, sg1, so0, so1):
    wid = lax.axis_index("s") * NUM_CORES + lax.axis_index("c")
    pltpu.sync_copy(idx_hbm.at[pl.ds(wid * IDX_PER_TILE, IDX_STAGE)], idx_v)

    def issue_gathers(c, buf, sem):
        for j in range(GATHERS_PER_CHUNK):
            pltpu.async_copy(
                table_hbm.at[
                    idx_v.at[pl.ds(c * ROWS_PER_CHUNK + j * GATHER_ROWS, GATHER_ROWS)]
                ],
                buf.at[pl.ds(j * GATHER_ROWS, GATHER_ROWS)],
                sem,
            )

    def wait_gathers(buf, sem):
        for j in range(GATHERS_PER_CHUNK):
            pltpu.make_async_copy(
                table_hbm.at[idx_v.at[pl.ds(j * GATHER_ROWS, GATHER_ROWS)]],
                buf.at[pl.ds(j * GATHER_ROWS, GATHER_ROWS)],
                sem,
            ).wait()

    def compute_chunk(c, rows_v, out_v, sem_o):
        @plsc.parallel_loop(0, CHUNK_FACES, unroll=2)
        def _face(f):
            r = 3 * f
            zero = jnp.zeros((LANES,), jnp.float32)
            acc01, acc02, acc21 = zero, zero, zero
            for g in range(GROUPS):
                sl = pl.ds(g * LANES, LANES)
                va = rows_v[r, sl]
                vb = rows_v[r + 1, sl]
                vc = rows_v[r + 2, sl]
                d01 = va - vb
                d02 = va - vc
                d21 = vc - vb
                acc01 = acc01 + d01 * d01
                acc02 = acc02 + d02 * d02
                acc21 = acc21 + d21 * d21
            out_v[f, pl.ds(0, LANES)] = acc01
            out_v[f, pl.ds(LANES, LANES)] = acc02
            out_v[f, pl.ds(2 * LANES, LANES)] = acc21

        out_base = wid * FACES_PER_TILE + c * CHUNK_FACES
        pltpu.async_copy(out_v, out_hbm.at[pl.ds(out_base, CHUNK_FACES)], sem_o)

    def wait_out(out_v, sem_o):
        pltpu.make_async_copy(
            out_v, out_hbm.at[pl.ds(0, CHUNK_FACES)], sem_o
        ).wait()

    issue_gathers(0, rows_v0, sg0)
    issue_gathers(1, rows_v1, sg1)

    def body(c2, carry):
        c = 2 * c2
        wait_gathers(rows_v0, sg0)

        @pl.when(c2 > 0)
        def _():
            wait_out(out_v0, so0)

        compute_chunk(c, rows_v0, out_v0, so0)
        issue_gathers(c + 2, rows_v0, sg0)
        wait_gathers(rows_v1, sg1)

        @pl.when(c2 > 0)
        def _():
            wait_out(out_v1, so1)

        compute_chunk(c + 1, rows_v1, out_v1, so1)
        issue_gathers(c + 3, rows_v1, sg1)
        return carry

    lax.fori_loop(0, N_CHUNKS // 2, body, 0)
    wait_gathers(rows_v0, sg0)
    wait_gathers(rows_v1, sg1)
    wait_out(out_v0, so0)
    wait_out(out_v1, so1)


_TC_BLK = 8192
_TC_GRID = NF_PAD // _TC_BLK


def _tc_reduce_body(x_ref, o_ref):
    i = pl.program_id(0)
    x = x_ref[...]
    d2a = jnp.sum(x[:, 0:LANES], axis=1)
    d2b = jnp.sum(x[:, LANES:2 * LANES], axis=1)
    d2c = jnp.sum(x[:, 2 * LANES:3 * LANES], axis=1)
    s = jnp.sum(jnp.sqrt(d2a)) + jnp.sum(jnp.sqrt(d2b)) + jnp.sum(jnp.sqrt(d2c))

    @pl.when(i == 0)
    def _init():
        o_ref[0, 0] = s

    @pl.when(i != 0)
    def _acc():
        o_ref[0, 0] = o_ref[0, 0] + s


_tc_reduce = pl.pallas_call(
    _tc_reduce_body,
    grid=(_TC_GRID,),
    in_specs=[pl.BlockSpec((_TC_BLK, 3 * LANES), lambda i: (i, 0))],
    out_specs=pl.BlockSpec(memory_space=pltpu.SMEM),
    out_shape=jax.ShapeDtypeStruct((1, 1), jnp.float32),
)


def kernel(vertex_features, faces):
    table = vertex_features.reshape(NV, D)
    flat_idx = faces.reshape(-1)
    pad = jnp.zeros((IDX_HBM_LEN - 3 * NF,), jnp.int32)
    idx_flat = jnp.concatenate([flat_idx, pad])
    partials = _sc_pair_partials(table, idx_flat)
    total = _tc_reduce(partials)
    return total.reshape(1)
